# Initial kernel scaffold; baseline (speedup 1.0000x reference)
import jax
import jax.numpy as jnp
from jax.experimental import pallas as pl


def kernel(x, edge_index, edge_attr, edge_type, w_self, w_h, w_t):
    raise NotImplementedError("write your pallas kernel here")


# speedup vs baseline: 2.4968x; 2.4968x over previous
import functools

import jax
import jax.numpy as jnp
from jax import lax
from jax.experimental import pallas as pl
from jax.experimental.pallas import tpu as pltpu
from jax.experimental.pallas import tpu_sc as plsc

N = 10000
E = 320000
D = 128

NC = 2
NS = 16
NW = NC * NS
CHUNK = 128
NCHUNKS = E // CHUNK
LANES = 16


def _mm_kernel(x_ref, wh_ref, wt_ref, h_ref, t_ref):
    xv = x_ref[...]
    h_ref[...] = 0.5 * jnp.dot(xv, wh_ref[...], preferred_element_type=jnp.float32)
    t_ref[...] = 0.5 * jnp.dot(xv, wt_ref[...], preferred_element_type=jnp.float32)


def _node_transform(x, w_h, w_t):
    blk = 1000
    return pl.pallas_call(
        _mm_kernel,
        grid=(N // blk,),
        in_specs=[
            pl.BlockSpec((blk, D), lambda i: (i, 0)),
            pl.BlockSpec((D, D), lambda i: (0, 0)),
            pl.BlockSpec((D, D), lambda i: (0, 0)),
        ],
        out_specs=[
            pl.BlockSpec((blk, D), lambda i: (i, 0)),
            pl.BlockSpec((blk, D), lambda i: (i, 0)),
        ],
        out_shape=[
            jax.ShapeDtypeStruct((N, D), jnp.float32),
            jax.ShapeDtypeStruct((N, D), jnp.float32),
        ],
    )(x, w_h, w_t)


@functools.cache
def _make_edge_kernel():
    mesh = plsc.VectorSubcoreMesh(core_axis_name="c", subcore_axis_name="s")

    @functools.partial(
        pl.kernel,
        mesh=mesh,
        out_type=jax.ShapeDtypeStruct((E, D), jnp.float32),
        scratch_types=[
            pltpu.VMEM((CHUNK,), jnp.int32),
            pltpu.VMEM((CHUNK,), jnp.int32),
            pltpu.VMEM((CHUNK, D), jnp.float32),
            pltpu.VMEM((CHUNK, D), jnp.float32),
            pltpu.SemaphoreType.DMA,
            pltpu.SemaphoreType.DMA,
        ],
    )
    def _edge_kernel(h_hbm, t_hbm, row_hbm, col_hbm, out_hbm,
                     idx_r, idx_c, hbuf, tbuf, sem_h, sem_t):
        wid = lax.axis_index("s") * NC + lax.axis_index("c")
        nchunks_w = (NCHUNKS - wid + NW - 1) // NW

        def chunk_body(k, carry):
            base = (wid + k * NW) * CHUNK
            pltpu.sync_copy(row_hbm.at[pl.ds(base, CHUNK)], idx_r)
            pltpu.sync_copy(col_hbm.at[pl.ds(base, CHUNK)], idx_c)
            cp_h = pltpu.async_copy(h_hbm.at[idx_r], hbuf, sem_h)
            cp_t = pltpu.async_copy(t_hbm.at[idx_c], tbuf, sem_t)
            cp_h.wait()
            cp_t.wait()

            def row_body(r, c2):
                for j in range(D // LANES):
                    sl = pl.ds(j * LANES, LANES)
                    hv = hbuf[r, sl]
                    tv = tbuf[r, sl]
                    hbuf[r, sl] = jnp.maximum(hv + tv, 0.0)
                return c2

            lax.fori_loop(0, CHUNK, row_body, 0, unroll=2)
            pltpu.sync_copy(hbuf, out_hbm.at[pl.ds(base, CHUNK)])
            return carry

        lax.fori_loop(0, nchunks_w, chunk_body, 0)

    return _edge_kernel


def kernel(x, edge_index, edge_attr, edge_type, w_self, w_h, w_t):
    del edge_attr, edge_type, w_self
    h, t = _node_transform(x, w_h, w_t)
    row = edge_index[0].astype(jnp.int32)
    col = edge_index[1].astype(jnp.int32)
    return _make_edge_kernel()(h, t, row, col)


# speedup vs baseline: 5.0390x; 2.0182x over previous
import functools

import jax
import jax.numpy as jnp
from jax import lax
from jax.experimental import pallas as pl
from jax.experimental.pallas import tpu as pltpu
from jax.experimental.pallas import tpu_sc as plsc

N = 10000
E = 320000
D = 128

NC = 2
NS = 16
NW = NC * NS
CHUNK = 128
NCHUNKS = E // CHUNK
LANES = 16


def _mm_kernel(x_ref, wh_ref, wt_ref, h_ref, t_ref):
    xv = x_ref[...]
    h_ref[...] = 0.5 * jnp.dot(xv, wh_ref[...], preferred_element_type=jnp.float32)
    t_ref[...] = 0.5 * jnp.dot(xv, wt_ref[...], preferred_element_type=jnp.float32)


def _node_transform(x, w_h, w_t):
    blk = 1000
    return pl.pallas_call(
        _mm_kernel,
        grid=(N // blk,),
        in_specs=[
            pl.BlockSpec((blk, D), lambda i: (i, 0)),
            pl.BlockSpec((D, D), lambda i: (0, 0)),
            pl.BlockSpec((D, D), lambda i: (0, 0)),
        ],
        out_specs=[
            pl.BlockSpec((blk, D), lambda i: (i, 0)),
            pl.BlockSpec((blk, D), lambda i: (i, 0)),
        ],
        out_shape=[
            jax.ShapeDtypeStruct((N, D), jnp.float32),
            jax.ShapeDtypeStruct((N, D), jnp.float32),
        ],
    )(x, w_h, w_t)


MAXC = -(-NCHUNKS // NW)
MAXC_EVEN = MAXC + (MAXC % 2)


@functools.cache
def _make_edge_kernel():
    mesh = plsc.VectorSubcoreMesh(core_axis_name="c", subcore_axis_name="s")

    @functools.partial(
        pl.kernel,
        mesh=mesh,
        out_type=jax.ShapeDtypeStruct((E, D), jnp.float32),
        scratch_types=[
            pltpu.VMEM((2, CHUNK), jnp.int32),
            pltpu.VMEM((2, CHUNK), jnp.int32),
            pltpu.VMEM((2, CHUNK, D), jnp.float32),
            pltpu.VMEM((2, CHUNK, D), jnp.float32),
            pltpu.VMEM((2, CHUNK, D), jnp.float32),
            pltpu.SemaphoreType.DMA,
            pltpu.SemaphoreType.DMA,
            pltpu.SemaphoreType.DMA,
            pltpu.SemaphoreType.DMA,
            pltpu.SemaphoreType.DMA,
            pltpu.SemaphoreType.DMA,
            pltpu.SemaphoreType.DMA,
            pltpu.SemaphoreType.DMA,
        ],
    )
    def _edge_kernel(h_hbm, t_hbm, row_hbm, col_hbm, out_hbm,
                     idxr, idxc, hb, tb, ob,
                     si0, si1, sh0, sh1, st0, st1, so0, so1):
        SI, SH, ST, SO = (si0, si1), (sh0, sh1), (st0, st1), (so0, so1)
        wid = lax.axis_index("s") * NC + lax.axis_index("c")
        n = (NCHUNKS - wid + NW - 1) // NW

        def cbase(c):
            return (wid + c * NW) * CHUNK

        def issue_idx(c, s):
            pltpu.async_copy(row_hbm.at[pl.ds(cbase(c), CHUNK)], idxr.at[s], SI[s])
            pltpu.async_copy(col_hbm.at[pl.ds(cbase(c), CHUNK)], idxc.at[s], SI[s])

        def wait_idx(s):
            pltpu.make_async_copy(row_hbm.at[pl.ds(0, CHUNK)], idxr.at[s], SI[s]).wait()
            pltpu.make_async_copy(col_hbm.at[pl.ds(0, CHUNK)], idxc.at[s], SI[s]).wait()

        def issue_gathers(s):
            pltpu.async_copy(h_hbm.at[idxr.at[s]], hb.at[s], SH[s])
            pltpu.async_copy(t_hbm.at[idxc.at[s]], tb.at[s], ST[s])

        def wait_gathers(s):
            pltpu.make_async_copy(h_hbm.at[idxr.at[s]], hb.at[s], SH[s]).wait()
            pltpu.make_async_copy(t_hbm.at[idxc.at[s]], tb.at[s], ST[s]).wait()

        def issue_writeout(c, s):
            pltpu.async_copy(ob.at[s], out_hbm.at[pl.ds(cbase(c), CHUNK)], SO[s])

        def wait_writeout(s):
            pltpu.make_async_copy(ob.at[s], out_hbm.at[pl.ds(0, CHUNK)], SO[s]).wait()

        def compute(s):
            def row_body(r, z):
                for j in range(D // LANES):
                    sl = pl.ds(j * LANES, LANES)
                    ob[s, r, sl] = jnp.maximum(hb[s, r, sl] + tb[s, r, sl], 0.0)
                return z
            lax.fori_loop(0, CHUNK, row_body, 0, unroll=2)

        issue_idx(0, 0)
        issue_idx(1, 1)
        wait_idx(0)
        issue_gathers(0)

        def pair_body(m, z):
            for half in range(2):
                c = m * 2 + half
                s = half

                @pl.when(c < n)
                def _():
                    wait_gathers(s)

                @pl.when(c + 2 < n)
                def _():
                    issue_idx(c + 2, s)

                @pl.when(c + 1 < n)
                def _():
                    wait_idx(1 - s)
                    issue_gathers(1 - s)

                @pl.when((c >= 2) & (c - 2 < n))
                def _():
                    wait_writeout(s)

                @pl.when(c < n)
                def _():
                    compute(s)
                    issue_writeout(c, s)
            return z

        lax.fori_loop(0, MAXC_EVEN // 2, pair_body, 0)

        for x in (MAXC_EVEN - 2, MAXC_EVEN - 1):
            @pl.when(x < n)
            def _():
                wait_writeout(x % 2)

    return _edge_kernel


def kernel(x, edge_index, edge_attr, edge_type, w_self, w_h, w_t):
    del edge_attr, edge_type, w_self
    h, t = _node_transform(x, w_h, w_t)
    row = edge_index[0].astype(jnp.int32)
    col = edge_index[1].astype(jnp.int32)
    return _make_edge_kernel()(h, t, row, col)
